# baseline (speedup 1.0000x reference)
import functools

import jax
import jax.numpy as jnp
import numpy as np
from jax import lax
from jax.experimental import pallas as pl
from jax.experimental.pallas import tpu as pltpu
from jax.experimental.pallas import tpu_sc as plsc

D_MODEL = 32
CHUNK = 40
NBUF = 2
SCALE = np.float32(np.sqrt(np.float32(D_MODEL)))

_NC = 2
_NS = 16
_NW = _NC * _NS


def _make_sc_kernel(n_seq: int, seq_len: int):
    assert n_seq % _NW == 0
    s_per_w = n_seq // _NW
    assert seq_len % CHUNK == 0
    chunks_per_row = seq_len // CHUNK
    n_chunks = s_per_w * chunks_per_row
    assert n_chunks % NBUF == 0

    mesh = plsc.VectorSubcoreMesh(core_axis_name="c", subcore_axis_name="s")

    @functools.partial(
        pl.kernel,
        mesh=mesh,
        out_type=jax.ShapeDtypeStruct((n_seq, seq_len, D_MODEL), jnp.float32),
        compiler_params=pltpu.CompilerParams(use_tc_tiling_on_sc=False),
        scratch_types=[
            pltpu.VMEM((s_per_w, seq_len), jnp.int32),
            pltpu.VMEM((NBUF, CHUNK, D_MODEL), jnp.float32),
            pltpu.VMEM((NBUF, CHUNK, D_MODEL), jnp.float32),
            pltpu.SemaphoreType.DMA,
            pltpu.SemaphoreType.DMA,
            pltpu.SemaphoreType.DMA,
            pltpu.SemaphoreType.DMA,
        ],
    )
    def k(idx_hbm, table_hbm, out_hbm, idx_v, gbuf, wbuf, gs0, gs1, ws0, ws1):
        gsems = (gs0, gs1)
        wsems = (ws0, ws1)
        wid = lax.axis_index("s") * _NC + lax.axis_index("c")
        s_base = wid * s_per_w

        pltpu.sync_copy(idx_hbm.at[pl.ds(s_base, s_per_w)], idx_v)

        def split(c):
            r = c // chunks_per_row
            t0 = (c - r * chunks_per_row) * CHUNK
            return r, t0

        def fire_gather(c, b):
            r, t0 = split(c)
            pltpu.async_copy(
                table_hbm.at[idx_v.at[r, pl.ds(t0, CHUNK)]], gbuf.at[b], gsems[b]
            )

        def fire_write(c, b):
            r, t0 = split(c)
            pltpu.async_copy(
                wbuf.at[b], out_hbm.at[s_base + r, pl.ds(t0, CHUNK)], wsems[b]
            )

        for b in range(NBUF):
            fire_gather(b, b)

        def body(c0, carry):
            for b in range(NBUF):
                c = c0 + b
                @pl.when(c0 >= NBUF)
                def _():
                    pltpu.make_async_copy(
                        wbuf.at[b], out_hbm.at[s_base, pl.ds(0, CHUNK)], wsems[b]
                    ).wait()

                pltpu.make_async_copy(
                    table_hbm.at[idx_v.at[0, pl.ds(0, CHUNK)]], gbuf.at[b], gsems[b]
                ).wait()

                for r in range(CHUNK):
                    for col in (0, 16):
                        wbuf[b, r, pl.ds(col, 16)] = gbuf[b, r, pl.ds(col, 16)] * SCALE

                fire_write(c, b)

                @pl.when(c0 + NBUF < n_chunks)
                def _():
                    fire_gather(c + NBUF, b)
            return carry

        lax.fori_loop(0, n_chunks // NBUF,
                      lambda i, cr: body(i * NBUF, cr), 0, unroll=False)

        for b in range(NBUF):
            pltpu.make_async_copy(
                wbuf.at[b], out_hbm.at[s_base, pl.ds(0, CHUNK)], wsems[b]
            ).wait()

    return k


def kernel(sequences, table):
    n_seq, seq_len = sequences.shape
    return _make_sc_kernel(n_seq, seq_len)(sequences, table)
